# baseline (speedup 1.0000x reference)
import functools

import jax
import jax.numpy as jnp
from jax import lax
from jax.experimental import pallas as pl
from jax.experimental.pallas import tpu as pltpu
from jax.experimental.pallas import tpu_sc as plsc

BS, N, LIE, DV, DE = 4, 1024, 6, 512, 4
M = 256
ALPHA = 0.2
SB = 512
NS = N // SB


def _fps_body(a_ref, x0, x1, x2, x3, x4, x5, chosen_ref, scratch):
    b = pl.program_id(0)
    s = pl.program_id(1)
    r = jnp.sqrt(x0[0] * x0[0] + x1[0] * x1[0] + x2[0] * x2[0])
    t = jnp.sqrt(x3[0] * x3[0] + x4[0] * x4[0] + x5[0] * x5[0])
    y = ALPHA * r + (1.0 - ALPHA) * t
    scratch[pl.ds((b * N + s * SB) * 8, SB * 8), :] = y.reshape(SB * 8, 128)

    @pl.when((b == BS - 1) & (s == NS - 1))
    def _():
        riota = jax.lax.broadcasted_iota(jnp.int32, (8, 128), 0)
        liota = jax.lax.broadcasted_iota(jnp.int32, (8, 128), 1)
        flat = riota * 128 + liota
        big = jnp.int32(2**30)

        def body(i, carry):
            ds_, fs = carry
            nds, nfs = [], []
            for b2 in range(BS):
                chosen_ref[b2, i] = fs[b2]
                row = scratch[pl.ds((b2 * N + fs[b2]) * 8, 8), :]
                d2 = jnp.minimum(ds_[b2], row)
                gmax = jnp.max(d2)
                cand = jnp.where(d2 == gmax, flat, big)
                nds.append(d2)
                nfs.append(jnp.min(cand))
            return tuple(nds), tuple(nfs)

        d0 = jnp.full((8, 128), 1e8, jnp.float32)
        jax.lax.fori_loop(
            0, M, body,
            ((d0,) * BS, tuple(a_ref[b2] for b2 in range(BS))))


def _fps_chosen(planes, interpret=False):
    a = jax.random.randint(jax.random.key(1), (BS,), 0, N).astype(jnp.int32)
    bspec = pl.BlockSpec((1, SB, N), lambda b, s: (b, s, 0))
    return pl.pallas_call(
        _fps_body,
        grid=(BS, NS),
        in_specs=[pl.BlockSpec(memory_space=pltpu.SMEM)] + [bspec] * LIE,
        out_specs=pl.BlockSpec(memory_space=pltpu.SMEM),
        out_shape=jax.ShapeDtypeStruct((BS, M), jnp.int32),
        scratch_shapes=[pltpu.VMEM((BS * N * 8, 128), jnp.float32)],
        interpret=interpret,
        compiler_params=pltpu.CompilerParams(
            dimension_semantics=("arbitrary", "arbitrary")),
    )(a, *planes)


NW = 32
RPW = BS * M // NW


def _sc_gather_rows(tbl, idxflat):
    d = tbl.shape[1]
    mesh = plsc.VectorSubcoreMesh(core_axis_name="c", subcore_axis_name="s")

    @functools.partial(
        pl.kernel, mesh=mesh,
        out_type=jax.ShapeDtypeStruct((BS * M, d), jnp.float32),
        scratch_types=[
            pltpu.VMEM((RPW,), jnp.int32),
            pltpu.VMEM((RPW, d), jnp.float32),
            pltpu.SemaphoreType.DMA,
        ],
    )
    def k(tbl_hbm, idx_hbm, out_hbm, idx_v, rows_v, sem):
        wid = lax.axis_index("s") * 2 + lax.axis_index("c")
        base = wid * RPW
        pltpu.sync_copy(idx_hbm.at[pl.ds(base, RPW)], idx_v)
        pltpu.async_copy(tbl_hbm.at[idx_v], rows_v, sem).wait()
        pltpu.sync_copy(rows_v, out_hbm.at[pl.ds(base, RPW)])

    return k(tbl, idxflat)


def kernel(abq_pairs, vals, mask, edges):
    planes = [abq_pairs[..., c] for c in range(LIE)]
    qidx = _fps_chosen(planes)
    B3 = jnp.arange(BS)[:, None, None]
    I3 = qidx[:, None, :]
    J3 = qidx[:, :, None]
    idxflat = (qidx + jnp.arange(BS, dtype=jnp.int32)[:, None] * N).reshape(BS * M)
    sub_abq = abq_pairs[B3, I3, J3]
    sub_vals = _sc_gather_rows(vals.reshape(BS * N, DV), idxflat).reshape(BS, M, DV)
    sub_mask = jnp.ones((BS, M), dtype=mask.dtype)
    sub_edges = edges[B3, I3, J3]
    return sub_abq, sub_vals, sub_mask, sub_edges


# speedup vs baseline: 2.5490x; 2.5490x over previous
import functools

import jax
import jax.numpy as jnp
from jax import lax
from jax.experimental import pallas as pl
from jax.experimental.pallas import tpu as pltpu
from jax.experimental.pallas import tpu_sc as plsc

BS, N, LIE, DV, DE = 4, 1024, 6, 512, 4
M = 256
ALPHA = 0.2
SB = 512
NS = N // SB


def _fps_body(a_ref, x0, x1, x2, x3, x4, x5, chosen_ref, scratch):
    b = pl.program_id(0)
    s = pl.program_id(1)
    r = jnp.sqrt(x0[0] * x0[0] + x1[0] * x1[0] + x2[0] * x2[0])
    t = jnp.sqrt(x3[0] * x3[0] + x4[0] * x4[0] + x5[0] * x5[0])
    y = ALPHA * r + (1.0 - ALPHA) * t
    scratch[pl.ds((b * N + s * SB) * 8, SB * 8), :] = y.reshape(SB * 8, 128)

    @pl.when((b == BS - 1) & (s == NS - 1))
    def _():
        riota = jax.lax.broadcasted_iota(jnp.int32, (8, 128), 0)
        liota = jax.lax.broadcasted_iota(jnp.int32, (8, 128), 1)
        flat = riota * 128 + liota
        big = jnp.int32(2**30)

        def body(i, carry):
            ds_, fs = carry
            nds, nfs = [], []
            for b2 in range(BS):
                chosen_ref[b2, i] = fs[b2]
                row = scratch[pl.ds((b2 * N + fs[b2]) * 8, 8), :]
                d2 = jnp.minimum(ds_[b2], row)
                gmax = jnp.max(d2)
                cand = jnp.where(d2 == gmax, flat, big)
                nds.append(d2)
                nfs.append(jnp.min(cand))
            return tuple(nds), tuple(nfs)

        d0 = jnp.full((8, 128), 1e8, jnp.float32)
        jax.lax.fori_loop(
            0, M, body,
            ((d0,) * BS, tuple(a_ref[b2] for b2 in range(BS))))


def _fps_chosen(planes, interpret=False):
    a = jax.random.randint(jax.random.key(1), (BS,), 0, N).astype(jnp.int32)
    bspec = pl.BlockSpec((1, SB, N), lambda b, s: (b, s, 0))
    return pl.pallas_call(
        _fps_body,
        grid=(BS, NS),
        in_specs=[pl.BlockSpec(memory_space=pltpu.SMEM)] + [bspec] * LIE,
        out_specs=pl.BlockSpec(memory_space=pltpu.SMEM),
        out_shape=jax.ShapeDtypeStruct((BS, M), jnp.int32),
        scratch_shapes=[pltpu.VMEM((BS * N * 8, 128), jnp.float32)],
        interpret=interpret,
        compiler_params=pltpu.CompilerParams(
            dimension_semantics=("arbitrary", "arbitrary")),
    )(a, *planes)


NW = 32
RPW = BS * M // NW


def _sc_gather_rows(tbl, idxflat):
    d = tbl.shape[1]
    mesh = plsc.VectorSubcoreMesh(core_axis_name="c", subcore_axis_name="s")

    @functools.partial(
        pl.kernel, mesh=mesh,
        out_type=jax.ShapeDtypeStruct((BS * M, d), jnp.float32),
        scratch_types=[
            pltpu.VMEM((RPW,), jnp.int32),
            pltpu.VMEM((RPW, d), jnp.float32),
            pltpu.SemaphoreType.DMA,
        ],
    )
    def k(tbl_hbm, idx_hbm, out_hbm, idx_v, rows_v, sem):
        wid = lax.axis_index("s") * 2 + lax.axis_index("c")
        base = wid * RPW
        pltpu.sync_copy(idx_hbm.at[pl.ds(base, RPW)], idx_v)
        pltpu.async_copy(tbl_hbm.at[idx_v], rows_v, sem).wait()
        pltpu.sync_copy(rows_v, out_hbm.at[pl.ds(base, RPW)])

    return k(tbl, idxflat)


def kernel(abq_pairs, vals, mask, edges):
    planes = [abq_pairs[..., c] for c in range(LIE)]
    qidx = _fps_chosen(planes)
    B = jnp.arange(BS)[:, None]
    idxflat = (qidx + jnp.arange(BS, dtype=jnp.int32)[:, None] * N).reshape(BS * M)
    sub_abq = abq_pairs[B, qidx][B, :, qidx]
    sub_vals = _sc_gather_rows(vals.reshape(BS * N, DV), idxflat).reshape(BS, M, DV)
    sub_mask = jnp.ones((BS, M), dtype=mask.dtype)
    sub_edges = edges[B, qidx][B, :, qidx]
    return sub_abq, sub_vals, sub_mask, sub_edges
